# baseline (speedup 1.0000x reference)
import functools

import jax
import jax.numpy as jnp
from jax import lax
from jax.experimental import pallas as pl
from jax.experimental.pallas import tpu as pltpu
from jax.experimental.pallas import tpu_sc as plsc

N = 10000
E = 160000
B = 8
U = 64
IN_DIM = 2
F = IN_DIM + U
NM = 3
CH = 128
NCH = 5
WIDE = NCH * CH

NC = 2
NS = 16
NW = NC * NS
EW = E // NW
K = 125
NB = EW // K
RT = N // NS

TE = 400

_f32 = jnp.float32


def _deg_body(src_hbm, zeros_hbm, ones_hbm, degp_hbm, idx_v, ones_v, acc):
    c = lax.axis_index("c")
    s = lax.axis_index("s")
    wid = c * NS + s
    base = s * RT
    pltpu.sync_copy(src_hbm.at[wid], idx_v)
    pltpu.sync_copy(ones_hbm, ones_v)
    pltpu.sync_copy(zeros_hbm, acc.at[pl.ds(base, RT)])
    plsc.subcore_barrier()

    def body(j, carry):
        pltpu.sync_copy(ones_v, acc.at[idx_v.at[j]], add=True)
        return carry

    lax.fori_loop(0, NB, body, 0)
    plsc.subcore_barrier()
    pltpu.sync_copy(acc.at[pl.ds(base, RT)], degp_hbm.at[c, s])


DW = 128


@functools.cache
def _deg_kernel():
    mesh = plsc.VectorSubcoreMesh(
        core_axis_name="c", subcore_axis_name="s", num_cores=NC, num_subcores=NS)
    return pl.kernel(
        _deg_body,
        out_type=jax.ShapeDtypeStruct((NC, NS, RT, DW), _f32),
        mesh=mesh,
        scratch_types=[
            pltpu.VMEM((NB, K), jnp.int32),
            pltpu.VMEM((K, DW), _f32),
            pltpu.VMEM_SHARED((N, DW), _f32),
        ],
    )


def _deg_call(src, zdeg, ones):
    return _deg_kernel()(src, zdeg, ones).reshape(NC, N, DW)[:, :, :1]


NB2 = E // NS // K


GID = 16
NG = NB2 // GID


def _make_spmv_body(nch_in, split4):
    def body(*refs):
        ys = refs[0:nch_in]
        src_hbm, dst_hbm, zeros_hbm = refs[nch_in:nch_in + 3]
        outs = refs[nch_in + 3:2 * nch_in + 3]
        (idxs_g, idxd_g, bufa, bufb, acc, sema, semb) = refs[2 * nch_in + 3:]
        c = lax.axis_index("c")
        s = lax.axis_index("s")
        base = s * RT

        def edge_loop(yref, g_lo, g_hi):
            def group(g, carry):
                pltpu.sync_copy(src_hbm.at[s, pl.ds(g * GID, GID)], idxs_g)
                pltpu.sync_copy(dst_hbm.at[s, pl.ds(g * GID, GID)], idxd_g)
                bufs = (bufa, bufb)
                sems = (sema, semb)
                pltpu.async_copy(yref.at[idxs_g.at[0]], bufa, sema)
                for j in range(GID):
                    cur, sem = bufs[j % 2], sems[j % 2]
                    if j + 1 < GID:
                        pltpu.async_copy(yref.at[idxs_g.at[j + 1]],
                                         bufs[(j + 1) % 2], sems[(j + 1) % 2])
                    pltpu.make_async_copy(yref.at[idxs_g.at[0]], cur, sem).wait()
                    pltpu.sync_copy(cur, acc.at[idxd_g.at[j]], add=True)
                return carry

            lax.fori_loop(g_lo, g_hi, group, 0)

        def run_chunk(j, g_lo, g_hi, out_slice):
            pltpu.sync_copy(zeros_hbm, bufa)
            for k in range(RT // K):
                pltpu.sync_copy(bufa, acc.at[pl.ds(base + k * K, K)])
            plsc.subcore_barrier()
            edge_loop(ys[j], g_lo, g_hi)
            plsc.subcore_barrier()
            pltpu.sync_copy(acc.at[pl.ds(base, RT)], out_slice)

        for c_id, chunk_list in ((0, (0, 1)), (1, (2, 3))):
            @pl.when(c == c_id)
            def _():
                for j in chunk_list:
                    run_chunk(j, 0, NG, outs[j].at[s])
                if split4:
                    g_lo, g_hi = (0, 3) if c_id == 0 else (3, NG)
                    run_chunk(4, g_lo, g_hi, outs[4].at[c_id, s])

    return body


@functools.lru_cache(maxsize=None)
def _spmv_kernel(nch_in):
    mesh = plsc.VectorSubcoreMesh(
        core_axis_name="c", subcore_axis_name="s", num_cores=NC, num_subcores=NS)
    out_type = [jax.ShapeDtypeStruct((NS, RT, CH), _f32)] * min(nch_in, 4)
    if nch_in == 5:
        out_type = out_type + [jax.ShapeDtypeStruct((NC, NS, RT, CH), _f32)]
    return pl.kernel(
        _make_spmv_body(nch_in, nch_in == 5),
        out_type=out_type,
        mesh=mesh,
        scratch_types=[
            pltpu.VMEM((GID, K), jnp.int32),
            pltpu.VMEM((GID, K), jnp.int32),
            pltpu.VMEM((K, CH), _f32),
            pltpu.VMEM((K, CH), _f32),
            pltpu.VMEM_SHARED((N, CH), _f32),
            pltpu.SemaphoreType.DMA,
            pltpu.SemaphoreType.DMA,
        ],
    )


def _spmv_call(ys, src, dst, zrow):
    n = len(ys)
    outs = _spmv_kernel(n)(*ys, src, dst, zrow)
    res = [o.reshape(N, CH) for o in outs[:4]]
    if n == 5:
        res.append(outs[4].reshape(NC, N, CH))
    return res


def _w_from_degp(degp):
    deg = degp[0, :, 0] + degp[1, :, 0]
    return 1.0 / jnp.maximum(deg, 1.0)


def _chunk_specs(n, block):
    return [pl.BlockSpec(block, lambda i: (i, 0)) for _ in range(n)]


def _feat(Xw, b):
    return jnp.concatenate(
        [Xw[:, b * U:(b + 1) * U],
         Xw[:, B * U + b * IN_DIM:B * U + (b + 1) * IN_DIM]], axis=1)


def _e0_body(degp, hf, it, *outs):
    w = _w_from_degp(degp)
    parts = []
    for b in range(B):
        hb = hf[b]
        h3b = jnp.concatenate([hb[:, None, :U], hb[:, None, U:]], axis=1)
        parts.append(h3b.reshape(TE, U))
    v2 = it[...].reshape(TE, IN_DIM, B)
    for b in range(B):
        for d_ in range(IN_DIM):
            parts.append(v2[:, d_, b][:, None])
    parts.append(jnp.zeros((TE, WIDE - B * F), _f32))
    wide = jnp.concatenate(parts, axis=1)
    yw = wide * w[:, None]
    for j in range(NCH):
        outs[j][...] = wide[:, j * CH:(j + 1) * CH]
        outs[NCH + j][...] = yw[:, j * CH:(j + 1) * CH]


def _e0_call(degp, hf, it):
    return pl.pallas_call(
        _e0_body,
        grid=(N // TE,),
        in_specs=[
            pl.BlockSpec((NC, TE, 1), lambda i: (0, i, 0)),
            pl.BlockSpec((B, TE // 2, CH), lambda i: (0, i, 0)),
            pl.BlockSpec((2 * TE, B), lambda i: (i, 0)),
        ],
        out_specs=_chunk_specs(2 * NCH, (TE, CH)),
        out_shape=[jax.ShapeDtypeStruct((N, CH), _f32)] * (2 * NCH),
    )(degp, hf, it)


def _make_scale_body(n, has4):
    def body(degp, *refs):
        ps = refs[:n]
        outs = refs[n:]
        w = _w_from_degp(degp)
        for j in range(n):
            if has4 and j == n - 1:
                xj = ps[j][0] + ps[j][1]
                outs[n][...] = xj
            else:
                xj = ps[j][...]
            outs[j][...] = xj * w[:, None]
    return body


def _scale_call(degp, parts, has4=False):
    n = len(parts)
    in_specs = ([pl.BlockSpec((NC, TE, 1), lambda i: (0, i, 0))]
                + [pl.BlockSpec((TE, CH), lambda i: (i, 0))] * (n - int(has4)))
    out_specs = _chunk_specs(n + int(has4), (TE, CH))
    if has4:
        in_specs = in_specs + [pl.BlockSpec((NC, TE, CH), lambda i: (0, i, 0))]
    return pl.pallas_call(
        _make_scale_body(n, has4),
        grid=(N // TE,),
        in_specs=in_specs,
        out_specs=out_specs,
        out_shape=[jax.ShapeDtypeStruct((N, CH), _f32)] * (n + int(has4)),
    )(degp, *parts)


def _wide_of(chunks):
    return jnp.concatenate([r[...] for r in chunks], axis=1)


def _m1_body(*refs):
    x0c = refs[0:NCH]
    x1c = refs[NCH:2 * NCH]
    p2c = refs[2 * NCH:3 * NCH]
    degp, wm, br = refs[3 * NCH:3 * NCH + 3]
    o = refs[3 * NCH + 3:]
    xc0_o = o[0:NCH]
    yc0_o = o[NCH:2 * NCH]
    g_o = o[2 * NCH]

    X0 = _wide_of(x0c)
    X1 = _wide_of(x1c)
    X2 = 2.0 * jnp.concatenate(
        [p[...] for p in p2c[:4]] + [p2c[4][0] + p2c[4][1]], axis=1) - X0
    w = _w_from_degp(degp)
    parts = []
    gs = []
    wcat = jnp.concatenate([wm[0], wm[1], wm[2]], axis=0)
    for b in range(B):
        xb = jnp.concatenate([_feat(X0, b), _feat(X1, b), _feat(X2, b)], axis=1)
        val = jnp.dot(xb, wcat, preferred_element_type=_f32) + br[0][None, :]
        g = jax.nn.sigmoid(val)
        gs.append(g[None])
        parts.append(g[:, :U] * X0[:, b * U:(b + 1) * U])
    parts.append(X0[:, B * U:B * F])
    parts.append(jnp.zeros((TE, WIDE - B * F), _f32))
    xc0w = jnp.concatenate(parts, axis=1)
    yc0w = xc0w * w[:, None]
    for j in range(NCH):
        xc0_o[j][...] = xc0w[:, j * CH:(j + 1) * CH]
        yc0_o[j][...] = yc0w[:, j * CH:(j + 1) * CH]
    g_o[...] = jnp.concatenate(gs, axis=0)


def _m1_call(x0c, x1c, p2c, degp, wm, br):
    return pl.pallas_call(
        _m1_body,
        grid=(N // TE,),
        in_specs=(
            [pl.BlockSpec((TE, CH), lambda i: (i, 0))] * (3 * NCH - 1)
            + [pl.BlockSpec((NC, TE, CH), lambda i: (0, i, 0))]
            + [
                pl.BlockSpec((NC, TE, 1), lambda i: (0, i, 0)),
                pl.BlockSpec((NM, F, 2 * U), lambda i: (0, 0, 0)),
                pl.BlockSpec((1, 2 * U), lambda i: (0, 0)),
            ]
        ),
        out_specs=(
            _chunk_specs(2 * NCH, (TE, CH))
            + [pl.BlockSpec((B, TE, 2 * U), lambda i: (0, i, 0))]
        ),
        out_shape=(
            [jax.ShapeDtypeStruct((N, CH), _f32)] * (2 * NCH)
            + [jax.ShapeDtypeStruct((B, N, 2 * U), _f32)]
        ),
    )(*x0c, *x1c, *p2c, degp, wm, br)


def _m2_body(*refs):
    xc0c = refs[0:NCH]
    xc1c = refs[NCH:2 * NCH]
    p4c = refs[2 * NCH:3 * NCH]
    x0c = refs[3 * NCH:4 * NCH]
    g, wm, bc = refs[4 * NCH:4 * NCH + 3]
    o_ref = refs[4 * NCH + 3]

    Xc0 = _wide_of(xc0c)
    Xc1 = _wide_of(xc1c)
    Xc2 = 2.0 * jnp.concatenate(
        [p[...] for p in p4c[:4]] + [p4c[4][0] + p4c[4][1]], axis=1) - Xc0
    X0 = _wide_of(x0c)
    rows = []
    wcat = jnp.concatenate([wm[0], wm[1], wm[2]], axis=0)
    for b in range(B):
        xb = jnp.concatenate(
            [_feat(Xc0, b), _feat(Xc1, b), _feat(Xc2, b)], axis=1)
        val = jnp.dot(xb, wcat, preferred_element_type=_f32) + bc[0][None, :]
        cand = jnp.tanh(val)
        u_b = g[b][:, U:]
        h_b = X0[:, b * U:(b + 1) * U]
        new_b = u_b * h_b + (1.0 - u_b) * cand
        nb = new_b.reshape(TE // 2, 2, U)
        rows.append(jnp.concatenate([nb[:, 0, :], nb[:, 1, :]], axis=1)[None])
    o_ref[...] = jnp.concatenate(rows, axis=0)


def _m2_call(xc0c, xc1c, p4c, x0c, g, wm, bc):
    return pl.pallas_call(
        _m2_body,
        grid=(N // TE,),
        in_specs=(
            [pl.BlockSpec((TE, CH), lambda i: (i, 0))] * (3 * NCH - 1)
            + [pl.BlockSpec((NC, TE, CH), lambda i: (0, i, 0))]
            + [pl.BlockSpec((TE, CH), lambda i: (i, 0))] * NCH
            + [
                pl.BlockSpec((B, TE, 2 * U), lambda i: (0, i, 0)),
                pl.BlockSpec((NM, F, U), lambda i: (0, 0, 0)),
                pl.BlockSpec((1, U), lambda i: (0, 0)),
            ]
        ),
        out_specs=pl.BlockSpec((B, TE // 2, CH), lambda i: (0, i, 0)),
        out_shape=jax.ShapeDtypeStruct((B, N // 2, CH), _f32),
    )(*xc0c, *xc1c, *p4c, *x0c, g, wm, bc)


def _w3(w, out_dim):
    wm = w.reshape(F, NM, out_dim).transpose(1, 0, 2)
    return jnp.concatenate([wm[:, IN_DIM:, :], wm[:, :IN_DIM, :]], axis=1)


@jax.jit
def kernel(inputs, hx, edge_index, W_ru, b_ru, W_c, b_c):
    hf = hx.reshape(B, N // 2, CH)
    it = inputs.T
    src = edge_index[0].reshape(NW, NB, K)
    dst = edge_index[1].reshape(NW, NB, K)
    wru = _w3(W_ru, 2 * U)
    wc = _w3(W_c, U)
    zrow = jnp.zeros((K, CH), _f32)
    zdeg = jnp.zeros((RT, DW), _f32)
    ones = jnp.ones((K, DW), _f32)

    src_s = edge_index[0].reshape(NS, NB2, K)
    dst_s = edge_index[1].reshape(NS, NB2, K)
    degp = _deg_call(src, zdeg, ones)
    e0 = _e0_call(degp, hf, it)
    x0c, y0c = e0[:NCH], e0[NCH:]
    p1 = _spmv_call(y0c, src_s, dst_s, zrow)
    s1 = _scale_call(degp, p1, has4=True)
    y1c, x1_4 = s1[:NCH], s1[NCH]
    x1c = p1[:4] + [x1_4]
    p2 = _spmv_call(y1c, src_s, dst_s, zrow)
    m1 = _m1_call(x0c, x1c, p2, degp, wru, b_ru.reshape(1, -1))
    xc0c, yc0c, g = m1[:NCH], m1[NCH:2 * NCH], m1[2 * NCH]
    p3 = _spmv_call(yc0c[:4], src_s, dst_s, zrow)
    xc1c = list(p3) + [x1_4]
    yc1b = _scale_call(degp, p3)
    p4 = _spmv_call(yc1b, src_s, dst_s, zrow)
    p4c = list(p4) + [p2[4]]
    out = _m2_call(xc0c, xc1c, p4c, x0c, g, wc, b_c.reshape(1, -1))
    return out.reshape(B, N * U)


# speedup vs baseline: 1.0134x; 1.0134x over previous
import functools

import jax
import jax.numpy as jnp
from jax import lax
from jax.experimental import pallas as pl
from jax.experimental.pallas import tpu as pltpu
from jax.experimental.pallas import tpu_sc as plsc

N = 10000
E = 160000
B = 8
U = 64
IN_DIM = 2
F = IN_DIM + U
NM = 3
CH = 128
NCH = 5
WIDE = NCH * CH

NC = 2
NS = 16
NW = NC * NS
EW = E // NW
K = 125
NB = EW // K
RT = N // NS

TE = 400

_f32 = jnp.float32


def _deg_body(src_hbm, zeros_hbm, ones_hbm, degp_hbm, idx_v, ones_v, acc):
    c = lax.axis_index("c")
    s = lax.axis_index("s")
    wid = c * NS + s
    base = s * RT
    pltpu.sync_copy(src_hbm.at[wid], idx_v)
    pltpu.sync_copy(ones_hbm, ones_v)
    pltpu.sync_copy(zeros_hbm, acc.at[pl.ds(base, RT)])
    plsc.subcore_barrier()

    def body(j, carry):
        pltpu.sync_copy(ones_v, acc.at[idx_v.at[j]], add=True)
        return carry

    lax.fori_loop(0, NB, body, 0)
    plsc.subcore_barrier()
    pltpu.sync_copy(acc.at[pl.ds(base, RT)], degp_hbm.at[c, s])


DW = 128


@functools.cache
def _deg_kernel():
    mesh = plsc.VectorSubcoreMesh(
        core_axis_name="c", subcore_axis_name="s", num_cores=NC, num_subcores=NS)
    return pl.kernel(
        _deg_body,
        out_type=jax.ShapeDtypeStruct((NC, NS, RT, DW), _f32),
        mesh=mesh,
        scratch_types=[
            pltpu.VMEM((NB, K), jnp.int32),
            pltpu.VMEM((K, DW), _f32),
            pltpu.VMEM_SHARED((N, DW), _f32),
        ],
    )


def _deg_call(src, zdeg, ones):
    return _deg_kernel()(src, zdeg, ones).reshape(NC, N, DW)[:, :, :1]


NB2 = E // NS // K


GID = 16
NG = NB2 // GID


def _make_spmv_body(nch_in, split4):
    def body(*refs):
        ys = refs[0:nch_in]
        src_hbm, dst_hbm, zeros_hbm = refs[nch_in:nch_in + 3]
        outs = refs[nch_in + 3:2 * nch_in + 3]
        (idxs_g, idxd_g, bufa, bufb, acc, sema, semb) = refs[2 * nch_in + 3:]
        c = lax.axis_index("c")
        s = lax.axis_index("s")
        base = s * RT

        def edge_loop(yref, g_lo, g_hi):
            def group(g, carry):
                pltpu.sync_copy(src_hbm.at[s, pl.ds(g * GID, GID)], idxs_g)
                pltpu.sync_copy(dst_hbm.at[s, pl.ds(g * GID, GID)], idxd_g)
                bufs = (bufa, bufb)
                sems = (sema, semb)
                pltpu.async_copy(yref.at[idxs_g.at[0]], bufa, sema)
                for j in range(GID):
                    cur, sem = bufs[j % 2], sems[j % 2]
                    if j + 1 < GID:
                        pltpu.async_copy(yref.at[idxs_g.at[j + 1]],
                                         bufs[(j + 1) % 2], sems[(j + 1) % 2])
                    pltpu.make_async_copy(yref.at[idxs_g.at[0]], cur, sem).wait()
                    pltpu.sync_copy(cur, acc.at[idxd_g.at[j]], add=True)
                return carry

            lax.fori_loop(g_lo, g_hi, group, 0)

        def run_chunk(j, g_lo, g_hi, out_slice):
            pltpu.sync_copy(zeros_hbm, bufa)
            for k in range(RT // K):
                pltpu.sync_copy(bufa, acc.at[pl.ds(base + k * K, K)])
            plsc.subcore_barrier()
            edge_loop(ys[j], g_lo, g_hi)
            plsc.subcore_barrier()
            pltpu.sync_copy(acc.at[pl.ds(base, RT)], out_slice)

        for c_id, chunk_list in ((0, (0, 1)), (1, (2, 3))):
            @pl.when(c == c_id)
            def _():
                for j in chunk_list:
                    run_chunk(j, 0, NG, outs[j].at[s])
                if split4:
                    g_lo, g_hi = (0, 3) if c_id == 0 else (3, NG)
                    run_chunk(4, g_lo, g_hi, outs[4].at[c_id, s])

    return body


@functools.lru_cache(maxsize=None)
def _spmv_kernel(nch_in):
    mesh = plsc.VectorSubcoreMesh(
        core_axis_name="c", subcore_axis_name="s", num_cores=NC, num_subcores=NS)
    out_type = [jax.ShapeDtypeStruct((NS, RT, CH), _f32)] * min(nch_in, 4)
    if nch_in == 5:
        out_type = out_type + [jax.ShapeDtypeStruct((NC, NS, RT, CH), _f32)]
    return pl.kernel(
        _make_spmv_body(nch_in, nch_in == 5),
        out_type=out_type,
        mesh=mesh,
        scratch_types=[
            pltpu.VMEM((GID, K), jnp.int32),
            pltpu.VMEM((GID, K), jnp.int32),
            pltpu.VMEM((K, CH), _f32),
            pltpu.VMEM((K, CH), _f32),
            pltpu.VMEM_SHARED((N, CH), _f32),
            pltpu.SemaphoreType.DMA,
            pltpu.SemaphoreType.DMA,
        ],
    )


def _spmv_call(ys, src, dst, zrow):
    n = len(ys)
    outs = _spmv_kernel(n)(*ys, src, dst, zrow)
    res = [o.reshape(N, CH) for o in outs[:4]]
    if n == 5:
        res.append(outs[4].reshape(NC, N, CH))
    return res


def _w_from_degp(degp):
    deg = degp[0, :, 0] + degp[1, :, 0]
    return 1.0 / jnp.maximum(deg, 1.0)


def _chunk_specs(n, block):
    return [pl.BlockSpec(block, lambda i: (i, 0)) for _ in range(n)]


def _feat(Xw, b):
    return jnp.concatenate(
        [Xw[:, b * U:(b + 1) * U],
         Xw[:, B * U + b * IN_DIM:B * U + (b + 1) * IN_DIM]], axis=1)


def _e0_body(degp, hf, it, *outs):
    w = _w_from_degp(degp)
    parts = []
    for b in range(B):
        hb = hf[b]
        h3b = jnp.concatenate([hb[:, None, :U], hb[:, None, U:]], axis=1)
        parts.append(h3b.reshape(TE, U))
    v2 = it[...].reshape(TE, IN_DIM, B)
    for b in range(B):
        for d_ in range(IN_DIM):
            parts.append(v2[:, d_, b][:, None])
    parts.append(jnp.zeros((TE, WIDE - B * F), _f32))
    wide = jnp.concatenate(parts, axis=1)
    yw = wide * w[:, None]
    for j in range(NCH):
        outs[j][...] = wide[:, j * CH:(j + 1) * CH]
        outs[NCH + j][...] = yw[:, j * CH:(j + 1) * CH]


def _e0_call(degp, hf, it):
    return pl.pallas_call(
        _e0_body,
        grid=(N // TE,),
        in_specs=[
            pl.BlockSpec((NC, TE, 1), lambda i: (0, i, 0)),
            pl.BlockSpec((B, TE // 2, CH), lambda i: (0, i, 0)),
            pl.BlockSpec((2 * TE, B), lambda i: (i, 0)),
        ],
        out_specs=_chunk_specs(2 * NCH, (TE, CH)),
        out_shape=[jax.ShapeDtypeStruct((N, CH), _f32)] * (2 * NCH),
    )(degp, hf, it)


def _make_scale_body(n, has4):
    def body(degp, *refs):
        ps = refs[:n]
        outs = refs[n:]
        w = _w_from_degp(degp)
        for j in range(n):
            if has4 and j == n - 1:
                xj = ps[j][0] + ps[j][1]
                outs[n][...] = xj
            else:
                xj = ps[j][...]
            outs[j][...] = xj * w[:, None]
    return body


def _scale_call(degp, parts, has4=False):
    n = len(parts)
    in_specs = ([pl.BlockSpec((NC, TE, 1), lambda i: (0, i, 0))]
                + [pl.BlockSpec((TE, CH), lambda i: (i, 0))] * (n - int(has4)))
    out_specs = _chunk_specs(n + int(has4), (TE, CH))
    if has4:
        in_specs = in_specs + [pl.BlockSpec((NC, TE, CH), lambda i: (0, i, 0))]
    return pl.pallas_call(
        _make_scale_body(n, has4),
        grid=(N // TE,),
        in_specs=in_specs,
        out_specs=out_specs,
        out_shape=[jax.ShapeDtypeStruct((N, CH), _f32)] * (n + int(has4)),
    )(degp, *parts)


def _wide_of(chunks):
    return jnp.concatenate([r[...] for r in chunks], axis=1)


def _m1_body(*refs):
    x0c = refs[0:NCH]
    x1c = refs[NCH:2 * NCH]
    p2c = refs[2 * NCH:3 * NCH]
    degp, wm, br = refs[3 * NCH:3 * NCH + 3]
    o = refs[3 * NCH + 3:]
    xc0_o = o[0:NCH]
    yc0_o = o[NCH:2 * NCH]
    g_o = o[2 * NCH]

    X0 = _wide_of(x0c)
    X1 = _wide_of(x1c)
    X2 = 2.0 * jnp.concatenate(
        [p[...] for p in p2c[:4]] + [p2c[4][0] + p2c[4][1]], axis=1) - X0
    w = _w_from_degp(degp)
    parts = []
    gs = []
    for b in range(B):
        val = (jnp.dot(_feat(X0, b), wm[0], preferred_element_type=_f32)
               + jnp.dot(_feat(X1, b), wm[1], preferred_element_type=_f32)
               + jnp.dot(_feat(X2, b), wm[2], preferred_element_type=_f32)
               + br[0][None, :])
        g = jax.nn.sigmoid(val)
        gs.append(g[None])
        parts.append(g[:, :U] * X0[:, b * U:(b + 1) * U])
    parts.append(X0[:, B * U:B * F])
    parts.append(jnp.zeros((TE, WIDE - B * F), _f32))
    xc0w = jnp.concatenate(parts, axis=1)
    yc0w = xc0w * w[:, None]
    for j in range(NCH):
        xc0_o[j][...] = xc0w[:, j * CH:(j + 1) * CH]
        yc0_o[j][...] = yc0w[:, j * CH:(j + 1) * CH]
    g_o[...] = jnp.concatenate(gs, axis=0)


def _m1_call(x0c, x1c, p2c, degp, wm, br):
    return pl.pallas_call(
        _m1_body,
        grid=(N // TE,),
        in_specs=(
            [pl.BlockSpec((TE, CH), lambda i: (i, 0))] * (3 * NCH - 1)
            + [pl.BlockSpec((NC, TE, CH), lambda i: (0, i, 0))]
            + [
                pl.BlockSpec((NC, TE, 1), lambda i: (0, i, 0)),
                pl.BlockSpec((NM, F, 2 * U), lambda i: (0, 0, 0)),
                pl.BlockSpec((1, 2 * U), lambda i: (0, 0)),
            ]
        ),
        out_specs=(
            _chunk_specs(2 * NCH, (TE, CH))
            + [pl.BlockSpec((B, TE, 2 * U), lambda i: (0, i, 0))]
        ),
        out_shape=(
            [jax.ShapeDtypeStruct((N, CH), _f32)] * (2 * NCH)
            + [jax.ShapeDtypeStruct((B, N, 2 * U), _f32)]
        ),
    )(*x0c, *x1c, *p2c, degp, wm, br)


def _m2_body(*refs):
    xc0c = refs[0:NCH]
    xc1c = refs[NCH:2 * NCH]
    p4c = refs[2 * NCH:3 * NCH]
    x0c = refs[3 * NCH:4 * NCH]
    g, wm, bc = refs[4 * NCH:4 * NCH + 3]
    o_ref = refs[4 * NCH + 3]

    Xc0 = _wide_of(xc0c)
    Xc1 = _wide_of(xc1c)
    Xc2 = 2.0 * jnp.concatenate(
        [p[...] for p in p4c[:4]] + [p4c[4][0] + p4c[4][1]], axis=1) - Xc0
    X0 = _wide_of(x0c)
    rows = []
    for b in range(B):
        val = (jnp.dot(_feat(Xc0, b), wm[0], preferred_element_type=_f32)
               + jnp.dot(_feat(Xc1, b), wm[1], preferred_element_type=_f32)
               + jnp.dot(_feat(Xc2, b), wm[2], preferred_element_type=_f32)
               + bc[0][None, :])
        cand = jnp.tanh(val)
        u_b = g[b][:, U:]
        h_b = X0[:, b * U:(b + 1) * U]
        new_b = u_b * h_b + (1.0 - u_b) * cand
        nb = new_b.reshape(TE // 2, 2, U)
        rows.append(jnp.concatenate([nb[:, 0, :], nb[:, 1, :]], axis=1)[None])
    o_ref[...] = jnp.concatenate(rows, axis=0)


def _m2_call(xc0c, xc1c, p4c, x0c, g, wm, bc):
    return pl.pallas_call(
        _m2_body,
        grid=(N // TE,),
        in_specs=(
            [pl.BlockSpec((TE, CH), lambda i: (i, 0))] * (3 * NCH - 1)
            + [pl.BlockSpec((NC, TE, CH), lambda i: (0, i, 0))]
            + [pl.BlockSpec((TE, CH), lambda i: (i, 0))] * NCH
            + [
                pl.BlockSpec((B, TE, 2 * U), lambda i: (0, i, 0)),
                pl.BlockSpec((NM, F, U), lambda i: (0, 0, 0)),
                pl.BlockSpec((1, U), lambda i: (0, 0)),
            ]
        ),
        out_specs=pl.BlockSpec((B, TE // 2, CH), lambda i: (0, i, 0)),
        out_shape=jax.ShapeDtypeStruct((B, N // 2, CH), _f32),
    )(*xc0c, *xc1c, *p4c, *x0c, g, wm, bc)


def _w3(w, out_dim):
    wm = w.reshape(F, NM, out_dim).transpose(1, 0, 2)
    return jnp.concatenate([wm[:, IN_DIM:, :], wm[:, :IN_DIM, :]], axis=1)


@jax.jit
def kernel(inputs, hx, edge_index, W_ru, b_ru, W_c, b_c):
    hf = hx.reshape(B, N // 2, CH)
    it = inputs.T
    src = edge_index[0].reshape(NW, NB, K)
    dst = edge_index[1].reshape(NW, NB, K)
    wru = _w3(W_ru, 2 * U)
    wc = _w3(W_c, U)
    zrow = jnp.zeros((K, CH), _f32)
    zdeg = jnp.zeros((RT, DW), _f32)
    ones = jnp.ones((K, DW), _f32)

    src_s = edge_index[0].reshape(NS, NB2, K)
    dst_s = edge_index[1].reshape(NS, NB2, K)
    degp = _deg_call(src, zdeg, ones)
    e0 = _e0_call(degp, hf, it)
    x0c, y0c = e0[:NCH], e0[NCH:]
    p1 = _spmv_call(y0c, src_s, dst_s, zrow)
    s1 = _scale_call(degp, p1, has4=True)
    y1c, x1_4 = s1[:NCH], s1[NCH]
    x1c = p1[:4] + [x1_4]
    p2 = _spmv_call(y1c, src_s, dst_s, zrow)
    m1 = _m1_call(x0c, x1c, p2, degp, wru, b_ru.reshape(1, -1))
    xc0c, yc0c, g = m1[:NCH], m1[NCH:2 * NCH], m1[2 * NCH]
    p3 = _spmv_call(yc0c[:4], src_s, dst_s, zrow)
    xc1c = list(p3) + [x1_4]
    yc1b = _scale_call(degp, p3)
    p4 = _spmv_call(yc1b, src_s, dst_s, zrow)
    p4c = list(p4) + [p2[4]]
    out = _m2_call(xc0c, xc1c, p4c, x0c, g, wc, b_c.reshape(1, -1))
    return out.reshape(B, N * U)
